# baseline (speedup 1.0000x reference)
import functools

import jax
import jax.numpy as jnp
from jax import lax
from jax.experimental import pallas as pl
from jax.experimental.pallas import tpu as pltpu
from jax.experimental.pallas import tpu_sc as plsc

_N = 10000
_D = 128
_E = 320000

_NC = 2
_NS = 16
_NW = _NC * _NS
_CHUNK = 128
_CHUNKS_PER_TILE = 80
_PER_TILE = _CHUNK * _CHUNKS_PER_TILE
_E_PAD = _PER_TILE * _NW
_ACC_ROWS = 10240
_ZERO_ROWS_PER_TILE = _ACC_ROWS // _NS
_OUT_ROWS_PER_TILE = _ACC_ROWS // _NS

_BN = 1000


def _lin_pair_first_body(h_ref, wl_ref, bl_ref, ws_ref, bs_ref, m_ref, s_ref):
    h = h_ref[...]
    m_ref[...] = jnp.dot(h, wl_ref[...], preferred_element_type=jnp.float32) + bl_ref[...]
    s_ref[...] = jnp.dot(h, ws_ref[...], preferred_element_type=jnp.float32) + bs_ref[...]


def _lin_pair_next_body(sp_ref, agg_ref, wl_ref, bl_ref, ws_ref, bs_ref, m_ref, s_ref):
    h = jnp.maximum(sp_ref[...] + agg_ref[0] + agg_ref[1], 0.0)
    m_ref[...] = jnp.dot(h, wl_ref[...], preferred_element_type=jnp.float32) + bl_ref[...]
    s_ref[...] = jnp.dot(h, ws_ref[...], preferred_element_type=jnp.float32) + bs_ref[...]


def _head_body(sp_ref, agg_ref, w1_ref, b1_ref, w2_ref, b2_ref, emb_ref, out_ref):
    emb = sp_ref[...] + agg_ref[0] + agg_ref[1]
    emb_ref[...] = emb
    h = jnp.maximum(emb, 0.0)
    t = jnp.maximum(
        jnp.dot(h, w1_ref[...], preferred_element_type=jnp.float32) + b1_ref[...], 0.0)
    out_ref[...] = jnp.dot(t, w2_ref[...], preferred_element_type=jnp.float32) + b2_ref[...]


_ROW_SPEC = pl.BlockSpec((_BN, _D), lambda i: (i, 0))
_W_SPEC = pl.BlockSpec((_D, _D), lambda i: (0, 0))
_B_SPEC = pl.BlockSpec((1, _D), lambda i: (0, 0))
_AGG_SPEC = pl.BlockSpec((_NC, _BN, _D), lambda i: (0, i, 0))
_GRID = (_N // _BN,)
_ND_OUT = jax.ShapeDtypeStruct((_N, _D), jnp.float32)


def _lin_pair_first(h, wl, bl, ws, bs):
    return pl.pallas_call(
        _lin_pair_first_body,
        grid=_GRID,
        in_specs=[_ROW_SPEC, _W_SPEC, _B_SPEC, _W_SPEC, _B_SPEC],
        out_specs=[_ROW_SPEC, _ROW_SPEC],
        out_shape=[_ND_OUT, _ND_OUT],
    )(h, wl, bl, ws, bs)


def _lin_pair_next(s_prev, agg, wl, bl, ws, bs):
    return pl.pallas_call(
        _lin_pair_next_body,
        grid=_GRID,
        in_specs=[_ROW_SPEC, _AGG_SPEC, _W_SPEC, _B_SPEC, _W_SPEC, _B_SPEC],
        out_specs=[_ROW_SPEC, _ROW_SPEC],
        out_shape=[_ND_OUT, _ND_OUT],
    )(s_prev, agg, wl, bl, ws, bs)


def _head(s_prev, agg, w1, b1, w2, b2):
    return pl.pallas_call(
        _head_body,
        grid=_GRID,
        in_specs=[_ROW_SPEC, _AGG_SPEC, _W_SPEC, _B_SPEC, _W_SPEC, _B_SPEC],
        out_specs=[_ROW_SPEC, _ROW_SPEC],
        out_shape=[_ND_OUT, _ND_OUT],
    )(s_prev, agg, w1, b1, w2, b2)


def _unpack_chunk(pidx_v, j, sidx_c, didx_c):
    for cc in range(_CHUNK // 16):
        v = pidx_v[j, pl.ds(cc * 16, 16)]
        sidx_c[pl.ds(cc * 16, 16)] = jnp.bitwise_and(v, 16383)
        didx_c[pl.ds(cc * 16, 16)] = lax.shift_right_logical(v, 14)


def _segsum_sc_body(m_hbm, packed_hbm, out_hbm,
                    pidx_v, sidx0, didx0, sidx1, didx1, rows0, rows1,
                    acc, sem0, sem1):
    c = lax.axis_index("c")
    s = lax.axis_index("s")
    wid = s * _NC + c

    zv = jnp.zeros((16,), jnp.float32)

    def zrow(r, carry):
        for cc in range(_D // 16):
            rows0[r, pl.ds(cc * 16, 16)] = zv
        return carry

    lax.fori_loop(0, _CHUNK, zrow, 0)
    zbase = s * _ZERO_ROWS_PER_TILE
    for z in range(_ZERO_ROWS_PER_TILE // _CHUNK):
        pltpu.sync_copy(rows0, acc.at[pl.ds(zbase + z * _CHUNK, _CHUNK)])
    plsc.subcore_barrier()

    pltpu.sync_copy(packed_hbm.at[wid], pidx_v)
    _unpack_chunk(pidx_v, 0, sidx0, didx0)
    pltpu.async_copy(m_hbm.at[sidx0], rows0, sem0)
    _unpack_chunk(pidx_v, 1, sidx1, didx1)

    def body(i, carry):
        j = 2 * i
        pltpu.async_copy(m_hbm.at[sidx1], rows1, sem1)
        pltpu.make_async_copy(m_hbm.at[sidx0], rows0, sem0).wait()
        pltpu.sync_copy(rows0, acc.at[didx0], add=True)

        @pl.when(j + 2 < _CHUNKS_PER_TILE)
        def _():
            _unpack_chunk(pidx_v, j + 2, sidx0, didx0)
            pltpu.async_copy(m_hbm.at[sidx0], rows0, sem0)

        pltpu.make_async_copy(m_hbm.at[sidx1], rows1, sem1).wait()
        pltpu.sync_copy(rows1, acc.at[didx1], add=True)

        @pl.when(j + 3 < _CHUNKS_PER_TILE)
        def _():
            _unpack_chunk(pidx_v, j + 3, sidx1, didx1)

        return carry

    lax.fori_loop(0, _CHUNKS_PER_TILE // 2, body, 0)
    plsc.subcore_barrier()

    obase = s * _OUT_ROWS_PER_TILE
    pltpu.sync_copy(acc.at[pl.ds(obase, _OUT_ROWS_PER_TILE)],
                    out_hbm.at[c, pl.ds(obase, _OUT_ROWS_PER_TILE)])


def _segsum(m, packed):
    mesh = plsc.VectorSubcoreMesh(core_axis_name="c", subcore_axis_name="s")
    fn = functools.partial(
        pl.kernel,
        mesh=mesh,
        out_type=jax.ShapeDtypeStruct((_NC, _ACC_ROWS, _D), jnp.float32),
        scratch_types=[
            pltpu.VMEM((_CHUNKS_PER_TILE, _CHUNK), jnp.int32),
            pltpu.VMEM((_CHUNK,), jnp.int32),
            pltpu.VMEM((_CHUNK,), jnp.int32),
            pltpu.VMEM((_CHUNK,), jnp.int32),
            pltpu.VMEM((_CHUNK,), jnp.int32),
            pltpu.VMEM((_CHUNK, _D), jnp.float32),
            pltpu.VMEM((_CHUNK, _D), jnp.float32),
            pltpu.VMEM_SHARED((_ACC_ROWS, _D), jnp.float32),
            pltpu.SemaphoreType.DMA,
            pltpu.SemaphoreType.DMA,
        ],
    )(_segsum_sc_body)
    return fn(m, packed)


def kernel(x, edge_index,
           W_lin0, b_lin0, W_self0, b_self0,
           W_lin1, b_lin1, W_self1, b_self1,
           W_lin2, b_lin2, W_self2, b_self2,
           W_mp1, b_mp1, W_mp2, b_mp2):
    src = edge_index[0]
    dst = edge_index[1]
    pad = _E_PAD - _E
    src_p = jnp.concatenate([src, jnp.zeros((pad,), jnp.int32)])
    dst_p = jnp.concatenate([dst, jnp.full((pad,), _N, jnp.int32)])
    packed = jnp.bitwise_or(jnp.left_shift(dst_p, 14), src_p)
    packed = packed.reshape(_NW, _CHUNKS_PER_TILE, _CHUNK)

    bl0 = b_lin0.reshape(1, _D)
    bs0 = b_self0.reshape(1, _D)
    bl1 = b_lin1.reshape(1, _D)
    bs1 = b_self1.reshape(1, _D)
    bl2 = b_lin2.reshape(1, _D)
    bs2 = b_self2.reshape(1, _D)
    bm1 = b_mp1.reshape(1, _D)
    bm2 = b_mp2.reshape(1, _D)

    m, s = _lin_pair_first(x, W_lin0, bl0, W_self0, bs0)
    agg = _segsum(m, packed)
    m, s = _lin_pair_next(s, agg, W_lin1, bl1, W_self1, bs1)
    agg = _segsum(m, packed)
    m, s = _lin_pair_next(s, agg, W_lin2, bl2, W_self2, bs2)
    agg = _segsum(m, packed)
    emb, out = _head(s, agg, W_mp1, bm1, W_mp2, bm2)
    return (emb, out)


# speedup vs baseline: 1.1003x; 1.1003x over previous
import functools

import jax
import jax.numpy as jnp
from jax import lax
from jax.experimental import pallas as pl
from jax.experimental.pallas import tpu as pltpu
from jax.experimental.pallas import tpu_sc as plsc

_N = 10000
_D = 128
_E = 320000

_NC = 2
_NS = 16
_NW = _NC * _NS
_CHUNK = 128
_K0 = 128
_K1 = 32
_SLOTS_PER_TILE = 128
_E0 = _NS * _K0 * _CHUNK
_E_PAD = _E0 + _NS * _K1 * _CHUNK
_ACC_ROWS = 10112
_ZERO_ROWS_PER_TILE = _ACC_ROWS // _NS
_OUT_ROWS_PER_TILE = _ACC_ROWS // _NS

_BN = 1000


def _lin_pair_first_body(h_ref, wl_ref, bl_ref, ws_ref, bs_ref, m_ref, s_ref):
    h = h_ref[...]
    m_ref[...] = jnp.dot(h, wl_ref[...], preferred_element_type=jnp.float32) + bl_ref[...]
    s_ref[...] = jnp.dot(h, ws_ref[...], preferred_element_type=jnp.float32) + bs_ref[...]


def _lin_pair_next_body(sp_ref, agg_ref, wl_ref, bl_ref, ws_ref, bs_ref, m_ref, s_ref):
    h = jnp.maximum(sp_ref[...] + agg_ref[0] + agg_ref[1], 0.0)
    m_ref[...] = jnp.dot(h, wl_ref[...], preferred_element_type=jnp.float32) + bl_ref[...]
    s_ref[...] = jnp.dot(h, ws_ref[...], preferred_element_type=jnp.float32) + bs_ref[...]


def _head_body(sp_ref, agg_ref, w1_ref, b1_ref, w2_ref, b2_ref, emb_ref, out_ref):
    emb = sp_ref[...] + agg_ref[0] + agg_ref[1]
    emb_ref[...] = emb
    h = jnp.maximum(emb, 0.0)
    t = jnp.maximum(
        jnp.dot(h, w1_ref[...], preferred_element_type=jnp.float32) + b1_ref[...], 0.0)
    out_ref[...] = jnp.dot(t, w2_ref[...], preferred_element_type=jnp.float32) + b2_ref[...]


_ROW_SPEC = pl.BlockSpec((_BN, _D), lambda i: (i, 0))
_W_SPEC = pl.BlockSpec((_D, _D), lambda i: (0, 0))
_B_SPEC = pl.BlockSpec((1, _D), lambda i: (0, 0))
_AGG_SPEC = pl.BlockSpec((_NC, _BN, _D), lambda i: (0, i, 0))
_GRID = (_N // _BN,)
_ND_OUT = jax.ShapeDtypeStruct((_N, _D), jnp.float32)


def _lin_pair_first(h, wl, bl, ws, bs):
    return pl.pallas_call(
        _lin_pair_first_body,
        grid=_GRID,
        in_specs=[_ROW_SPEC, _W_SPEC, _B_SPEC, _W_SPEC, _B_SPEC],
        out_specs=[_ROW_SPEC, _ROW_SPEC],
        out_shape=[_ND_OUT, _ND_OUT],
    )(h, wl, bl, ws, bs)


def _lin_pair_next(s_prev, agg, wl, bl, ws, bs):
    return pl.pallas_call(
        _lin_pair_next_body,
        grid=_GRID,
        in_specs=[_ROW_SPEC, _AGG_SPEC, _W_SPEC, _B_SPEC, _W_SPEC, _B_SPEC],
        out_specs=[_ROW_SPEC, _ROW_SPEC],
        out_shape=[_ND_OUT, _ND_OUT],
    )(s_prev, agg, wl, bl, ws, bs)


def _head(s_prev, agg, w1, b1, w2, b2):
    return pl.pallas_call(
        _head_body,
        grid=_GRID,
        in_specs=[_ROW_SPEC, _AGG_SPEC, _W_SPEC, _B_SPEC, _W_SPEC, _B_SPEC],
        out_specs=[_ROW_SPEC, _ROW_SPEC],
        out_shape=[_ND_OUT, _ND_OUT],
    )(s_prev, agg, w1, b1, w2, b2)


def _unpack_chunk(pidx_v, j, sidx_c, didx_c):
    for cc in range(_CHUNK // 16):
        v = pidx_v[j, pl.ds(cc * 16, 16)]
        sidx_c[pl.ds(cc * 16, 16)] = jnp.bitwise_and(v, 16383)
        didx_c[pl.ds(cc * 16, 16)] = lax.shift_right_logical(v, 14)


def _segsum_sc_body(m_hbm, packed_hbm, out_hbm,
                    pidx_v, sidx0, didx0, sidx1, didx1, rows0, rows1,
                    acc, sem0, sem1):
    c = lax.axis_index("c")
    s = lax.axis_index("s")
    wid = c * _NS + s
    nch = jnp.where(c == 0, _K0, _K1)

    zv = jnp.zeros((16,), jnp.float32)

    def zrow(r, carry):
        for cc in range(_D // 16):
            rows0[r, pl.ds(cc * 16, 16)] = zv
        return carry

    lax.fori_loop(0, _CHUNK, zrow, 0)
    zbase = s * _ZERO_ROWS_PER_TILE
    for z in range(_ZERO_ROWS_PER_TILE // _CHUNK):
        pltpu.sync_copy(rows0, acc.at[pl.ds(zbase + z * _CHUNK, _CHUNK)])
    rem = _ZERO_ROWS_PER_TILE % _CHUNK
    if rem:
        pltpu.sync_copy(
            rows0.at[pl.ds(0, rem)],
            acc.at[pl.ds(zbase + _ZERO_ROWS_PER_TILE - rem, rem)])
    plsc.subcore_barrier()

    pltpu.sync_copy(packed_hbm.at[wid], pidx_v)
    _unpack_chunk(pidx_v, 0, sidx0, didx0)
    pltpu.async_copy(m_hbm.at[sidx0], rows0, sem0)
    _unpack_chunk(pidx_v, 1, sidx1, didx1)

    def body(i, carry):
        j = 2 * i
        pltpu.async_copy(m_hbm.at[sidx1], rows1, sem1)
        pltpu.make_async_copy(m_hbm.at[sidx0], rows0, sem0).wait()
        pltpu.sync_copy(rows0, acc.at[didx0], add=True)

        @pl.when(j + 2 < nch)
        def _():
            _unpack_chunk(pidx_v, j + 2, sidx0, didx0)
            pltpu.async_copy(m_hbm.at[sidx0], rows0, sem0)

        pltpu.make_async_copy(m_hbm.at[sidx1], rows1, sem1).wait()
        pltpu.sync_copy(rows1, acc.at[didx1], add=True)

        @pl.when(j + 3 < nch)
        def _():
            _unpack_chunk(pidx_v, j + 3, sidx1, didx1)

        return carry

    lax.fori_loop(0, nch // 2, body, 0)
    plsc.subcore_barrier()

    obase = s * _OUT_ROWS_PER_TILE
    pltpu.sync_copy(acc.at[pl.ds(obase, _OUT_ROWS_PER_TILE)],
                    out_hbm.at[c, pl.ds(obase, _OUT_ROWS_PER_TILE)])


def _segsum(m, packed):
    mesh = plsc.VectorSubcoreMesh(core_axis_name="c", subcore_axis_name="s")
    fn = functools.partial(
        pl.kernel,
        mesh=mesh,
        out_type=jax.ShapeDtypeStruct((_NC, _ACC_ROWS, _D), jnp.float32),
        scratch_types=[
            pltpu.VMEM((_SLOTS_PER_TILE, _CHUNK), jnp.int32),
            pltpu.VMEM((_CHUNK,), jnp.int32),
            pltpu.VMEM((_CHUNK,), jnp.int32),
            pltpu.VMEM((_CHUNK,), jnp.int32),
            pltpu.VMEM((_CHUNK,), jnp.int32),
            pltpu.VMEM((_CHUNK, _D), jnp.float32),
            pltpu.VMEM((_CHUNK, _D), jnp.float32),
            pltpu.VMEM_SHARED((_ACC_ROWS, _D), jnp.float32),
            pltpu.SemaphoreType.DMA,
            pltpu.SemaphoreType.DMA,
        ],
    )(_segsum_sc_body)
    return fn(m, packed)


def kernel(x, edge_index,
           W_lin0, b_lin0, W_self0, b_self0,
           W_lin1, b_lin1, W_self1, b_self1,
           W_lin2, b_lin2, W_self2, b_self2,
           W_mp1, b_mp1, W_mp2, b_mp2):
    src = edge_index[0]
    dst = edge_index[1]
    pad = _E_PAD - _E
    src_p = jnp.concatenate([src, jnp.zeros((pad,), jnp.int32)])
    dst_p = jnp.concatenate([dst, jnp.full((pad,), _N, jnp.int32)])
    packed_flat = jnp.bitwise_or(jnp.left_shift(dst_p, 14), src_p)
    e0 = packed_flat[:_E0].reshape(_NS, _K0, _CHUNK)
    e1 = packed_flat[_E0:].reshape(_NS, _K1, _CHUNK)
    slotpad = jnp.full((_NS, _SLOTS_PER_TILE - _K1, _CHUNK), _N << 14,
                       dtype=jnp.int32)
    e1 = jnp.concatenate([e1, slotpad], axis=1)
    packed = jnp.concatenate([e0, e1], axis=0)

    bl0 = b_lin0.reshape(1, _D)
    bs0 = b_self0.reshape(1, _D)
    bl1 = b_lin1.reshape(1, _D)
    bs1 = b_self1.reshape(1, _D)
    bl2 = b_lin2.reshape(1, _D)
    bs2 = b_self2.reshape(1, _D)
    bm1 = b_mp1.reshape(1, _D)
    bm2 = b_mp2.reshape(1, _D)

    m, s = _lin_pair_first(x, W_lin0, bl0, W_self0, bs0)
    agg = _segsum(m, packed)
    m, s = _lin_pair_next(s, agg, W_lin1, bl1, W_self1, bs1)
    agg = _segsum(m, packed)
    m, s = _lin_pair_next(s, agg, W_lin2, bl2, W_self2, bs2)
    agg = _segsum(m, packed)
    emb, out = _head(s, agg, W_mp1, bm1, W_mp2, bm2)
    return (emb, out)


# speedup vs baseline: 1.1063x; 1.0054x over previous
import functools

import jax
import jax.numpy as jnp
from jax import lax
from jax.experimental import pallas as pl
from jax.experimental.pallas import tpu as pltpu
from jax.experimental.pallas import tpu_sc as plsc

_N = 10000
_D = 128
_E = 320000

_NC = 2
_NS = 16
_NW = _NC * _NS
_CHUNK = 128
_K0 = 128
_K1 = 32
_SLOTS_PER_TILE = 128
_E0 = _NS * _K0 * _CHUNK
_E_PAD = _E0 + _NS * _K1 * _CHUNK
_ACC_ROWS = 10112
_ZERO_ROWS_PER_TILE = _ACC_ROWS // _NS
_OUT_ROWS_PER_TILE = _ACC_ROWS // _NS

_BN = 1000


def _lin_pair_first_body(h_ref, wl_ref, bl_ref, ws_ref, bs_ref, m_ref, s_ref):
    h = h_ref[...]
    m_ref[...] = jnp.dot(h, wl_ref[...], preferred_element_type=jnp.float32) + bl_ref[...]
    s_ref[...] = jnp.dot(h, ws_ref[...], preferred_element_type=jnp.float32) + bs_ref[...]


def _lin_pair_next_body(sp_ref, agg_ref, wl_ref, bl_ref, ws_ref, bs_ref, m_ref, s_ref):
    h = jnp.maximum(sp_ref[...] + agg_ref[0] + agg_ref[1], 0.0)
    m_ref[...] = jnp.dot(h, wl_ref[...], preferred_element_type=jnp.float32) + bl_ref[...]
    s_ref[...] = jnp.dot(h, ws_ref[...], preferred_element_type=jnp.float32) + bs_ref[...]


def _head_body(sp_ref, agg_ref, w1_ref, b1_ref, w2_ref, b2_ref, emb_ref, out_ref):
    emb = sp_ref[...] + agg_ref[0] + agg_ref[1]
    emb_ref[...] = emb
    h = jnp.maximum(emb, 0.0)
    t = jnp.maximum(
        jnp.dot(h, w1_ref[...], preferred_element_type=jnp.float32) + b1_ref[...], 0.0)
    out_ref[...] = jnp.dot(t, w2_ref[...], preferred_element_type=jnp.float32) + b2_ref[...]


_ROW_SPEC = pl.BlockSpec((_BN, _D), lambda i: (i, 0))
_W_SPEC = pl.BlockSpec((_D, _D), lambda i: (0, 0))
_B_SPEC = pl.BlockSpec((1, _D), lambda i: (0, 0))
_AGG_SPEC = pl.BlockSpec((_NC, _BN, _D), lambda i: (0, i, 0))
_GRID = (_N // _BN,)
_ND_OUT = jax.ShapeDtypeStruct((_N, _D), jnp.float32)


def _lin_pair_first(h, wl, bl, ws, bs):
    return pl.pallas_call(
        _lin_pair_first_body,
        grid=_GRID,
        in_specs=[_ROW_SPEC, _W_SPEC, _B_SPEC, _W_SPEC, _B_SPEC],
        out_specs=[_ROW_SPEC, _ROW_SPEC],
        out_shape=[_ND_OUT, _ND_OUT],
    )(h, wl, bl, ws, bs)


def _lin_pair_next(s_prev, agg, wl, bl, ws, bs):
    return pl.pallas_call(
        _lin_pair_next_body,
        grid=_GRID,
        in_specs=[_ROW_SPEC, _AGG_SPEC, _W_SPEC, _B_SPEC, _W_SPEC, _B_SPEC],
        out_specs=[_ROW_SPEC, _ROW_SPEC],
        out_shape=[_ND_OUT, _ND_OUT],
    )(s_prev, agg, wl, bl, ws, bs)


def _head(s_prev, agg, w1, b1, w2, b2):
    return pl.pallas_call(
        _head_body,
        grid=_GRID,
        in_specs=[_ROW_SPEC, _AGG_SPEC, _W_SPEC, _B_SPEC, _W_SPEC, _B_SPEC],
        out_specs=[_ROW_SPEC, _ROW_SPEC],
        out_shape=[_ND_OUT, _ND_OUT],
    )(s_prev, agg, w1, b1, w2, b2)


def _unpack_chunk(pidx_v, j, sidx_c, didx_c):
    for cc in range(_CHUNK // 16):
        v = pidx_v[j, pl.ds(cc * 16, 16)]
        sidx_c[pl.ds(cc * 16, 16)] = jnp.bitwise_and(v, 16383)
        didx_c[pl.ds(cc * 16, 16)] = lax.shift_right_logical(v, 14)


def _segsum_sc_body(m_hbm, packed_hbm, out_hbm,
                    pidx_v, sidx0, didx0, sidx1, didx1, rows0, rows1,
                    acc, sem0, sem1):
    c = lax.axis_index("c")
    s = lax.axis_index("s")
    wid = c * _NS + s
    nch = jnp.where(c == 0, _K0, _K1)

    zv = jnp.zeros((16,), jnp.float32)

    def zrow(r, carry):
        for cc in range(_D // 16):
            rows0[r, pl.ds(cc * 16, 16)] = zv
        return carry

    with jax.named_scope("zero_vec"):
        lax.fori_loop(0, _CHUNK, zrow, 0)
    with jax.named_scope("zero_acc"):
        zbase = s * _ZERO_ROWS_PER_TILE
        for z in range(_ZERO_ROWS_PER_TILE // _CHUNK):
            pltpu.sync_copy(rows0, acc.at[pl.ds(zbase + z * _CHUNK, _CHUNK)])
        rem = _ZERO_ROWS_PER_TILE % _CHUNK
        if rem:
            pltpu.sync_copy(
                rows0.at[pl.ds(0, rem)],
                acc.at[pl.ds(zbase + _ZERO_ROWS_PER_TILE - rem, rem)])
    with jax.named_scope("zbar"):
        plsc.subcore_barrier()

    with jax.named_scope("stage_idx"):
        pltpu.sync_copy(packed_hbm.at[wid], pidx_v)
        _unpack_chunk(pidx_v, 0, sidx0, didx0)
        pltpu.async_copy(m_hbm.at[sidx0], rows0, sem0)
        _unpack_chunk(pidx_v, 1, sidx1, didx1)

    def body(i, carry):
        j = 2 * i
        pltpu.async_copy(m_hbm.at[sidx1], rows1, sem1)
        pltpu.make_async_copy(m_hbm.at[sidx0], rows0, sem0).wait()
        pltpu.sync_copy(rows0, acc.at[didx0], add=True)

        @pl.when(j + 2 < nch)
        def _():
            _unpack_chunk(pidx_v, j + 2, sidx0, didx0)
            pltpu.async_copy(m_hbm.at[sidx0], rows0, sem0)

        pltpu.make_async_copy(m_hbm.at[sidx1], rows1, sem1).wait()
        pltpu.sync_copy(rows1, acc.at[didx1], add=True)

        @pl.when(j + 3 < nch)
        def _():
            _unpack_chunk(pidx_v, j + 3, sidx1, didx1)

        return carry

    with jax.named_scope("mainloop"):
        lax.fori_loop(0, nch // 2, body, 0)
    with jax.named_scope("endbar"):
        plsc.subcore_barrier()

    with jax.named_scope("writeback"):
        obase = s * _OUT_ROWS_PER_TILE
        pltpu.sync_copy(acc.at[pl.ds(obase, _OUT_ROWS_PER_TILE)],
                        out_hbm.at[c, pl.ds(obase, _OUT_ROWS_PER_TILE)])


def _segsum(m, packed):
    mesh = plsc.VectorSubcoreMesh(core_axis_name="c", subcore_axis_name="s")
    fn = functools.partial(
        pl.kernel,
        mesh=mesh,
        out_type=jax.ShapeDtypeStruct((_NC, _ACC_ROWS, _D), jnp.float32),
        scratch_types=[
            pltpu.VMEM((_SLOTS_PER_TILE, _CHUNK), jnp.int32),
            pltpu.VMEM((_CHUNK,), jnp.int32),
            pltpu.VMEM((_CHUNK,), jnp.int32),
            pltpu.VMEM((_CHUNK,), jnp.int32),
            pltpu.VMEM((_CHUNK,), jnp.int32),
            pltpu.VMEM((_CHUNK, _D), jnp.float32),
            pltpu.VMEM((_CHUNK, _D), jnp.float32),
            pltpu.VMEM_SHARED((_ACC_ROWS, _D), jnp.float32),
            pltpu.SemaphoreType.DMA,
            pltpu.SemaphoreType.DMA,
        ],
    )(_segsum_sc_body)
    return fn(m, packed)


def kernel(x, edge_index,
           W_lin0, b_lin0, W_self0, b_self0,
           W_lin1, b_lin1, W_self1, b_self1,
           W_lin2, b_lin2, W_self2, b_self2,
           W_mp1, b_mp1, W_mp2, b_mp2):
    src = edge_index[0]
    dst = edge_index[1]
    pad = _E_PAD - _E
    src_p = jnp.concatenate([src, jnp.zeros((pad,), jnp.int32)])
    dst_p = jnp.concatenate([dst, jnp.full((pad,), _N, jnp.int32)])
    packed_flat = jnp.bitwise_or(jnp.left_shift(dst_p, 14), src_p)
    e0 = packed_flat[:_E0].reshape(_NS, _K0, _CHUNK)
    e1 = packed_flat[_E0:].reshape(_NS, _K1, _CHUNK)
    slotpad = jnp.full((_NS, _SLOTS_PER_TILE - _K1, _CHUNK), _N << 14,
                       dtype=jnp.int32)
    e1 = jnp.concatenate([e1, slotpad], axis=1)
    packed = jnp.concatenate([e0, e1], axis=0)

    bl0 = b_lin0.reshape(1, _D)
    bs0 = b_self0.reshape(1, _D)
    bl1 = b_lin1.reshape(1, _D)
    bs1 = b_self1.reshape(1, _D)
    bl2 = b_lin2.reshape(1, _D)
    bs2 = b_self2.reshape(1, _D)
    bm1 = b_mp1.reshape(1, _D)
    bm2 = b_mp2.reshape(1, _D)

    m, s = _lin_pair_first(x, W_lin0, bl0, W_self0, bs0)
    agg = _segsum(m, packed)
    m, s = _lin_pair_next(s, agg, W_lin1, bl1, W_self1, bs1)
    agg = _segsum(m, packed)
    m, s = _lin_pair_next(s, agg, W_lin2, bl2, W_self2, bs2)
    agg = _segsum(m, packed)
    emb, out = _head(s, agg, W_mp1, bm1, W_mp2, bm2)
    return (emb, out)
